# baseline (speedup 1.0000x reference)
import functools

import jax
import jax.numpy as jnp
from jax import lax
from jax.experimental import pallas as pl
from jax.experimental.pallas import tpu as pltpu
from jax.experimental.pallas import tpu_sc as plsc

_L = 16
_TW = 128
_NW = 32
_VPT = _TW // _L


def _wid():
  return lax.axis_index("s") * 2 + lax.axis_index("c")


def _block_sizes(tps, tpb):
  sizes = [tpb] * (tps // tpb)
  if tps % tpb:
    sizes.append(tps % tpb)
  return sizes


def _stream_blocks(srcs, bufs2, sems2, base, sizes, process):
  nsrc = len(srcs)
  depths = [len(bufs2[k]) for k in range(nsrc)]
  offs = [0]
  for sz in sizes[:-1]:
    offs.append(offs[-1] + sz)
  cps = [[None] * depths[k] for k in range(nsrc)]

  def start(k, b):
    slot = b % depths[k]
    sz = sizes[b]
    dst = bufs2[k][slot]
    if sz != dst.shape[0]:
      dst = dst.at[pl.ds(0, sz)]
    cps[k][slot] = pltpu.async_copy(
        srcs[k].at[pl.ds(base + offs[b], sz)], dst, sems2[k][slot])

  for k in range(nsrc):
    for b in range(min(depths[k], len(sizes))):
      start(k, b)

  def run(carry):
    for b, sz in enumerate(sizes):
      for k in range(nsrc):
        cps[k][b % depths[k]].wait()
      carry = process(
          [bufs2[k][b % depths[k]] for k in range(nsrc)], sz, carry)
      for k in range(nsrc):
        if b + depths[k] < len(sizes):
          start(k, b + depths[k])
    return carry

  return run


def _minmax_body(n_genes, tps, rem, tpb, num_tiles,
                 tgt, out, b0, b1, tail, stmin, stmax, sem0, sem1, semt):
  wid = _wid()
  base = wid * tps + jnp.minimum(wid, rem)

  tailcp = pltpu.async_copy(
      tgt.at[pl.ds(jnp.minimum(base + tps, num_tiles - 1), 1)], tail, semt)

  inf = jnp.full((_L,), jnp.inf, jnp.float32)
  ninf = jnp.full((_L,), -jnp.inf, jnp.float32)

  def process(bufs, sz, carry):
    buf = bufs[0]

    def body(i, carry):
      ti = i >> 3
      j = (i & 7) * _L
      new = []
      for g in range(n_genes):
        t = buf[ti, g, pl.ds(j, _L)]
        new.append(jnp.minimum(carry[g], t))
        new.append(jnp.maximum(carry[n_genes + g], t))
      return tuple(new[0::2]) + tuple(new[1::2])

    return lax.fori_loop(0, sz * _VPT, body, carry, unroll=2)

  run = _stream_blocks([tgt], [(b0, b1)], [(sem0, sem1)], base,
                       _block_sizes(tps, tpb), process)
  carry = run((inf,) * n_genes + (ninf,) * n_genes)
  mins = list(carry[:n_genes])
  maxs = list(carry[n_genes:])

  tailcp.wait()
  pick = jnp.broadcast_to(wid < rem, (_L,))
  for g in range(n_genes):
    for j in range(_VPT):
      t = tail[0, g, pl.ds(j * _L, _L)]
      mins[g] = jnp.minimum(mins[g], jnp.where(pick, t, inf))
      maxs[g] = jnp.maximum(maxs[g], jnp.where(pick, t, ninf))

  for g in range(n_genes):
    stmin[g, :] = mins[g]
    stmax[g, :] = maxs[g]
  pltpu.sync_copy(stmin, out.at[wid, 0])
  pltpu.sync_copy(stmax, out.at[wid, 1])


def _wmse_body(n_genes, kbins, tps, rem, tpb, num_tiles,
               pred, tgt, wts, minmax, out,
               tb0, tb1, tb2, pb0, pb1, ttail, ptail, wv, mmv, stacc,
               ts0, ts1, ts2, ps0, ps1, tst, pst):
  wid = _wid()
  base = wid * tps + jnp.minimum(wid, rem)

  tailidx = jnp.minimum(base + tps, num_tiles - 1)
  ttcp = pltpu.async_copy(tgt.at[pl.ds(tailidx, 1)], ttail, tst)
  ptcp = pltpu.async_copy(pred.at[pl.ds(tailidx, 1)], ptail, pst)

  pltpu.sync_copy(wts, wv)
  pltpu.sync_copy(minmax, mmv)

  inf = jnp.full((_L,), jnp.inf, jnp.float32)
  ninf = jnp.full((_L,), -jnp.inf, jnp.float32)

  def fold(w, carry):
    new = []
    for g in range(n_genes):
      new.append(jnp.minimum(carry[g], mmv[w, 0, g, pl.ds(0, _L)]))
      new.append(jnp.maximum(carry[n_genes + g], mmv[w, 1, g, pl.ds(0, _L)]))
    return tuple(new[0::2]) + tuple(new[1::2])

  folded = lax.fori_loop(0, _NW, fold, (inf,) * n_genes + (ninf,) * n_genes)
  m = []
  s = []
  kvec = jnp.full((_L,), float(kbins), jnp.float32)
  for g in range(n_genes):
    mnv = jnp.full((_L,), jnp.min(folded[g]), jnp.float32)
    mxv = jnp.full((_L,), jnp.max(folded[n_genes + g]), jnp.float32)
    m.append(mnv)
    s.append(kvec / (mxv - mnv))

  zero = jnp.zeros((_L,), jnp.float32)
  kmaxf = jnp.full((_L,), float(kbins - 1), jnp.float32)
  wrows = [wv.at[g] for g in range(n_genes)]

  def process(bufs, sz, accs):
    tbuf, pbuf = bufs

    def body(i, accs):
      ti = i >> 3
      j = (i & 7) * _L
      new = []
      for g in range(n_genes):
        t = tbuf[ti, g, pl.ds(j, _L)]
        p = pbuf[ti, g, pl.ds(j, _L)]
        d = p - t
        u = jnp.minimum((t - m[g]) * s[g], kmaxf)
        w = plsc.load_gather(wrows[g], [u.astype(jnp.int32)])
        new.append(accs[g] + d * d * w)
      return tuple(new)

    return lax.fori_loop(0, sz * _VPT, body, accs, unroll=2)

  run = _stream_blocks([tgt, pred], [(tb0, tb1, tb2), (pb0, pb1)],
                       [(ts0, ts1, ts2), (ps0, ps1)], base,
                       _block_sizes(tps, tpb), process)
  accs = list(run((zero,) * n_genes))

  ttcp.wait()
  ptcp.wait()
  pick = jnp.broadcast_to(wid < rem, (_L,))
  for g in range(n_genes):
    for j in range(_VPT):
      t = ttail[0, g, pl.ds(j * _L, _L)]
      p = ptail[0, g, pl.ds(j * _L, _L)]
      d = p - t
      u = jnp.minimum((t - m[g]) * s[g], kmaxf)
      w = plsc.load_gather(wrows[g], [u.astype(jnp.int32)])
      accs[g] = accs[g] + jnp.where(pick, d * d * w, zero)

  for g in range(n_genes):
    stacc[g, :] = accs[g]
  pltpu.sync_copy(stacc, out.at[wid])


@jax.jit
def kernel(pred, target, weights):
  n, n_genes = target.shape
  kbins = weights.shape[1]
  num_tiles = n // _TW
  tps, rem = divmod(num_tiles, _NW)
  tpb1 = 61 if tps % 61 == 0 else 32
  tpb2 = 32

  mesh = plsc.VectorSubcoreMesh(core_axis_name="c", subcore_axis_name="s")
  t3 = target.T.reshape(n_genes, num_tiles, _TW).transpose(1, 0, 2)
  p3 = pred.T.reshape(n_genes, num_tiles, _TW).transpose(1, 0, 2)

  minmax = pl.kernel(
      functools.partial(_minmax_body, n_genes, tps, rem, tpb1, num_tiles),
      out_type=jax.ShapeDtypeStruct((_NW, 2, n_genes, _L), jnp.float32),
      mesh=mesh,
      scratch_types=[
          pltpu.VMEM((tpb1, n_genes, _TW), jnp.float32),
          pltpu.VMEM((tpb1, n_genes, _TW), jnp.float32),
          pltpu.VMEM((1, n_genes, _TW), jnp.float32),
          pltpu.VMEM((n_genes, _L), jnp.float32),
          pltpu.VMEM((n_genes, _L), jnp.float32),
          pltpu.SemaphoreType.DMA,
          pltpu.SemaphoreType.DMA,
          pltpu.SemaphoreType.DMA,
      ],
      compiler_params=pltpu.CompilerParams(needs_layout_passes=False),
  )(t3)

  partial = pl.kernel(
      functools.partial(
          _wmse_body, n_genes, kbins, tps, rem, tpb2, num_tiles),
      out_type=jax.ShapeDtypeStruct((_NW, n_genes, _L), jnp.float32),
      mesh=mesh,
      scratch_types=[
          pltpu.VMEM((tpb2, n_genes, _TW), jnp.float32),
          pltpu.VMEM((tpb2, n_genes, _TW), jnp.float32),
          pltpu.VMEM((tpb2, n_genes, _TW), jnp.float32),
          pltpu.VMEM((tpb2, n_genes, _TW), jnp.float32),
          pltpu.VMEM((tpb2, n_genes, _TW), jnp.float32),
          pltpu.VMEM((1, n_genes, _TW), jnp.float32),
          pltpu.VMEM((1, n_genes, _TW), jnp.float32),
          pltpu.VMEM((n_genes, kbins), jnp.float32),
          pltpu.VMEM((_NW, 2, n_genes, _L), jnp.float32),
          pltpu.VMEM((n_genes, _L), jnp.float32),
          pltpu.SemaphoreType.DMA,
          pltpu.SemaphoreType.DMA,
          pltpu.SemaphoreType.DMA,
          pltpu.SemaphoreType.DMA,
          pltpu.SemaphoreType.DMA,
          pltpu.SemaphoreType.DMA,
          pltpu.SemaphoreType.DMA,
      ],
      compiler_params=pltpu.CompilerParams(needs_layout_passes=False),
  )(p3, t3, weights, minmax)

  return jnp.sum(partial) / (n_genes * n)


# speedup vs baseline: 1.0048x; 1.0048x over previous
import functools

import jax
import jax.numpy as jnp
from jax import lax
from jax.experimental import pallas as pl
from jax.experimental.pallas import tpu as pltpu
from jax.experimental.pallas import tpu_sc as plsc

_L = 16
_TW = 128
_NW = 32
_VPT = _TW // _L


def _wid():
  return lax.axis_index("s") * 2 + lax.axis_index("c")


def _block_sizes(tps, tpb):
  sizes = [tpb] * (tps // tpb)
  if tps % tpb:
    sizes.append(tps % tpb)
  return sizes


def _stream_blocks(srcs, bufs2, sems2, base, sizes, process):
  nsrc = len(srcs)
  depths = [len(bufs2[k]) for k in range(nsrc)]
  offs = [0]
  for sz in sizes[:-1]:
    offs.append(offs[-1] + sz)
  cps = [[None] * depths[k] for k in range(nsrc)]

  def start(k, b):
    slot = b % depths[k]
    sz = sizes[b]
    dst = bufs2[k][slot]
    if sz != dst.shape[0]:
      dst = dst.at[pl.ds(0, sz)]
    cps[k][slot] = pltpu.async_copy(
        srcs[k].at[pl.ds(base + offs[b], sz)], dst, sems2[k][slot])

  for k in range(nsrc):
    for b in range(min(depths[k], len(sizes))):
      start(k, b)

  def run(carry):
    for b, sz in enumerate(sizes):
      for k in range(nsrc):
        cps[k][b % depths[k]].wait()
      carry = process(
          [bufs2[k][b % depths[k]] for k in range(nsrc)], sz, carry)
      for k in range(nsrc):
        if b + depths[k] < len(sizes):
          start(k, b + depths[k])
    return carry

  return run


def _minmax_body(n_genes, tps, rem, tpb, num_tiles,
                 tgt, out, b0, b1, tail, stmin, stmax, sem0, sem1, semt):
  wid = _wid()
  base = wid * tps + jnp.minimum(wid, rem)

  tailcp = pltpu.async_copy(
      tgt.at[pl.ds(jnp.minimum(base + tps, num_tiles - 1), 1)], tail, semt)

  inf = jnp.full((_L,), jnp.inf, jnp.float32)
  ninf = jnp.full((_L,), -jnp.inf, jnp.float32)

  def process(bufs, sz, carry):
    buf = bufs[0]

    def body(i, carry):
      ti = i >> 3
      j = (i & 7) * _L
      new = []
      for g in range(n_genes):
        t = buf[ti, g, pl.ds(j, _L)]
        new.append(jnp.minimum(carry[g], t))
        new.append(jnp.maximum(carry[n_genes + g], t))
      return tuple(new[0::2]) + tuple(new[1::2])

    return lax.fori_loop(0, sz * _VPT, body, carry, unroll=2)

  run = _stream_blocks([tgt], [(b0, b1)], [(sem0, sem1)], base,
                       _block_sizes(tps, tpb), process)
  carry = run((inf,) * n_genes + (ninf,) * n_genes)
  mins = list(carry[:n_genes])
  maxs = list(carry[n_genes:])

  tailcp.wait()
  pick = jnp.broadcast_to(wid < rem, (_L,))
  for g in range(n_genes):
    for j in range(_VPT):
      t = tail[0, g, pl.ds(j * _L, _L)]
      mins[g] = jnp.minimum(mins[g], jnp.where(pick, t, inf))
      maxs[g] = jnp.maximum(maxs[g], jnp.where(pick, t, ninf))

  for g in range(n_genes):
    stmin[g, :] = mins[g]
    stmax[g, :] = maxs[g]
  pltpu.sync_copy(stmin, out.at[wid, 0])
  pltpu.sync_copy(stmax, out.at[wid, 1])


def _wmse_body(n_genes, kbins, tps, rem, tpb, num_tiles,
               pred, tgt, wts, minmax, out,
               tb0, tb1, pb0, pb1, ttail, ptail, wv, mmv, stacc,
               ts0, ts1, ps0, ps1, tst, pst):
  wid = _wid()
  base = wid * tps + jnp.minimum(wid, rem)

  tailidx = jnp.minimum(base + tps, num_tiles - 1)
  ttcp = pltpu.async_copy(tgt.at[pl.ds(tailidx, 1)], ttail, tst)
  ptcp = pltpu.async_copy(pred.at[pl.ds(tailidx, 1)], ptail, pst)

  pltpu.sync_copy(wts, wv)
  pltpu.sync_copy(minmax, mmv)

  inf = jnp.full((_L,), jnp.inf, jnp.float32)
  ninf = jnp.full((_L,), -jnp.inf, jnp.float32)

  def fold(w, carry):
    new = []
    for g in range(n_genes):
      new.append(jnp.minimum(carry[g], mmv[w, 0, g, pl.ds(0, _L)]))
      new.append(jnp.maximum(carry[n_genes + g], mmv[w, 1, g, pl.ds(0, _L)]))
    return tuple(new[0::2]) + tuple(new[1::2])

  folded = lax.fori_loop(0, _NW, fold, (inf,) * n_genes + (ninf,) * n_genes)
  m = []
  s = []
  kvec = jnp.full((_L,), float(kbins), jnp.float32)
  for g in range(n_genes):
    mnv = jnp.full((_L,), jnp.min(folded[g]), jnp.float32)
    mxv = jnp.full((_L,), jnp.max(folded[n_genes + g]), jnp.float32)
    m.append(mnv)
    s.append(kvec / (mxv - mnv))

  zero = jnp.zeros((_L,), jnp.float32)
  kmaxf = jnp.full((_L,), float(kbins - 1), jnp.float32)
  wrows = [wv.at[g] for g in range(n_genes)]

  def process(bufs, sz, accs):
    tbuf, pbuf = bufs

    def body(i, accs):
      ti = i >> 3
      j = (i & 7) * _L
      new = []
      for g in range(n_genes):
        t = tbuf[ti, g, pl.ds(j, _L)]
        p = pbuf[ti, g, pl.ds(j, _L)]
        d = p - t
        u = jnp.minimum((t - m[g]) * s[g], kmaxf)
        w = plsc.load_gather(wrows[g], [u.astype(jnp.int32)])
        new.append(accs[g] + d * d * w)
      return tuple(new)

    return lax.fori_loop(0, sz * _VPT, body, accs, unroll=2)

  run = _stream_blocks([tgt, pred], [(tb0, tb1), (pb0, pb1)],
                       [(ts0, ts1), (ps0, ps1)], base,
                       _block_sizes(tps, tpb), process)
  accs = list(run((zero,) * n_genes))

  ttcp.wait()
  ptcp.wait()
  pick = jnp.broadcast_to(wid < rem, (_L,))
  for g in range(n_genes):
    for j in range(_VPT):
      t = ttail[0, g, pl.ds(j * _L, _L)]
      p = ptail[0, g, pl.ds(j * _L, _L)]
      d = p - t
      u = jnp.minimum((t - m[g]) * s[g], kmaxf)
      w = plsc.load_gather(wrows[g], [u.astype(jnp.int32)])
      accs[g] = accs[g] + jnp.where(pick, d * d * w, zero)

  for g in range(n_genes):
    stacc[g, :] = accs[g]
  pltpu.sync_copy(stacc, out.at[wid])


@jax.jit
def kernel(pred, target, weights):
  n, n_genes = target.shape
  kbins = weights.shape[1]
  num_tiles = n // _TW
  tps, rem = divmod(num_tiles, _NW)
  tpb1 = 61 if tps % 61 == 0 else 32
  tpb2 = 32

  mesh = plsc.VectorSubcoreMesh(core_axis_name="c", subcore_axis_name="s")
  t3 = target.T.reshape(n_genes, num_tiles, _TW).transpose(1, 0, 2)
  p3 = pred.T.reshape(n_genes, num_tiles, _TW).transpose(1, 0, 2)

  minmax = pl.kernel(
      functools.partial(_minmax_body, n_genes, tps, rem, tpb1, num_tiles),
      out_type=jax.ShapeDtypeStruct((_NW, 2, n_genes, _L), jnp.float32),
      mesh=mesh,
      scratch_types=[
          pltpu.VMEM((tpb1, n_genes, _TW), jnp.float32),
          pltpu.VMEM((tpb1, n_genes, _TW), jnp.float32),
          pltpu.VMEM((1, n_genes, _TW), jnp.float32),
          pltpu.VMEM((n_genes, _L), jnp.float32),
          pltpu.VMEM((n_genes, _L), jnp.float32),
          pltpu.SemaphoreType.DMA,
          pltpu.SemaphoreType.DMA,
          pltpu.SemaphoreType.DMA,
      ],
      compiler_params=pltpu.CompilerParams(needs_layout_passes=False),
  )(t3)

  partial = pl.kernel(
      functools.partial(
          _wmse_body, n_genes, kbins, tps, rem, tpb2, num_tiles),
      out_type=jax.ShapeDtypeStruct((_NW, n_genes, _L), jnp.float32),
      mesh=mesh,
      scratch_types=[
          pltpu.VMEM((tpb2, n_genes, _TW), jnp.float32),
          pltpu.VMEM((tpb2, n_genes, _TW), jnp.float32),
          pltpu.VMEM((tpb2, n_genes, _TW), jnp.float32),
          pltpu.VMEM((tpb2, n_genes, _TW), jnp.float32),
          pltpu.VMEM((1, n_genes, _TW), jnp.float32),
          pltpu.VMEM((1, n_genes, _TW), jnp.float32),
          pltpu.VMEM((n_genes, kbins), jnp.float32),
          pltpu.VMEM((_NW, 2, n_genes, _L), jnp.float32),
          pltpu.VMEM((n_genes, _L), jnp.float32),
          pltpu.SemaphoreType.DMA,
          pltpu.SemaphoreType.DMA,
          pltpu.SemaphoreType.DMA,
          pltpu.SemaphoreType.DMA,
          pltpu.SemaphoreType.DMA,
          pltpu.SemaphoreType.DMA,
      ],
      compiler_params=pltpu.CompilerParams(needs_layout_passes=False),
  )(p3, t3, weights, minmax)

  return jnp.sum(partial) / (n_genes * n)
